# Initial kernel scaffold; baseline (speedup 1.0000x reference)
import jax
import jax.numpy as jnp
from jax.experimental import pallas as pl


def kernel(node_attributes, edge_attributes, global_attributes, edge_index, W, b):
    raise NotImplementedError("write your pallas kernel here")


# speedup vs baseline: 7.5291x; 7.5291x over previous
import functools

import jax
import jax.numpy as jnp
from jax import lax
from jax.experimental import pallas as pl
from jax.experimental.pallas import tpu as pltpu
from jax.experimental.pallas import tpu_sc as plsc

N_NODES = 100000
N_EDGES = 3200000
D_EDGE = 16
D_NODE = 128
D_GLOBAL = 32

_NS = 16
_IDXW = 128
_NROWS = N_EDGES // _IDXW
_CH = 8
_CHH = _CH // 2
_CHE = _CHH * _IDXW


def _rows_lo(s):
    return ((s * _NROWS) // _NS) // 8 * 8


def _nodes_lo(s):
    return (s * (N_NODES // 8)) // _NS * 8


def _seg_body(edges_hbm, dst_hbm, src_hbm, rec_hbm, sen_hbm, rows_v, idx_v, acc_sh):
    c = lax.axis_index("c")
    s = lax.axis_index("s")

    z = jnp.zeros((16,), jnp.float32)

    def zero_row(i, carry):
        rows_v[i, :] = z
        return carry

    lax.fori_loop(0, _CHE, zero_row, 0)
    nlo = _nodes_lo(s)
    span = _nodes_lo(s + 1) - nlo

    def zero_big(p, carry):
        pltpu.sync_copy(rows_v, acc_sh.at[pl.ds(nlo + p * _CHE, _CHE)])
        return carry

    nbig = span // _CHE
    lax.fori_loop(0, nbig, zero_big, 0)

    def zero_small(p, carry):
        pltpu.sync_copy(rows_v.at[pl.ds(0, 8)],
                        acc_sh.at[pl.ds(nlo + nbig * _CHE + p * 8, 8)])
        return carry

    lax.fori_loop(0, (span - nbig * _CHE) // 8, zero_small, 0)
    plsc.subcore_barrier()

    lo = _rows_lo(s)
    hi = _rows_lo(s + 1)

    def accumulate(idx_hbm):
        def chunk(k, carry):
            r0 = lo + k * _CH
            pltpu.sync_copy(idx_hbm.at[pl.ds(r0, _CH)], idx_v)
            for h in range(2):
                pltpu.sync_copy(
                    edges_hbm.at[pl.ds((r0 + h * _CHH) * _IDXW, _CHE)], rows_v)
                for j in range(_CHH):
                    pltpu.sync_copy(rows_v.at[pl.ds(j * _IDXW, _IDXW)],
                                    acc_sh.at[idx_v.at[h * _CHH + j]], add=True)
            return carry

        lax.fori_loop(0, (hi - lo) // _CH, chunk, 0)

    @pl.when(c == 0)
    def _():
        accumulate(dst_hbm)

    @pl.when(c == 1)
    def _():
        accumulate(src_hbm)

    plsc.subcore_barrier()

    def write_out(out_hbm):
        def out_big(p, carry):
            o = nlo + p * _CHE
            pltpu.sync_copy(acc_sh.at[pl.ds(o, _CHE)], out_hbm.at[pl.ds(o, _CHE)])
            return carry

        lax.fori_loop(0, nbig, out_big, 0)

        def out_small(p, carry):
            o = nlo + nbig * _CHE + p * 8
            pltpu.sync_copy(acc_sh.at[pl.ds(o, 8)], out_hbm.at[pl.ds(o, 8)])
            return carry

        lax.fori_loop(0, (span - nbig * _CHE) // 8, out_small, 0)

    @pl.when(c == 0)
    def _():
        write_out(rec_hbm)

    @pl.when(c == 1)
    def _():
        write_out(sen_hbm)


_seg = functools.partial(
    pl.kernel,
    out_type=[jax.ShapeDtypeStruct((N_NODES, D_EDGE), jnp.float32),
              jax.ShapeDtypeStruct((N_NODES, D_EDGE), jnp.float32)],
    mesh=plsc.VectorSubcoreMesh(core_axis_name="c", subcore_axis_name="s"),
    scratch_types=[
        pltpu.VMEM((_CHE, D_EDGE), jnp.float32),
        pltpu.VMEM((_CH, _IDXW), jnp.int32),
        pltpu.VMEM_SHARED((N_NODES, D_EDGE), jnp.float32),
    ],
    compiler_params=pltpu.CompilerParams(use_tc_tiling_on_sc=False),
)(_seg_body)


_BM = 2000


def _mm_body(rec_ref, sen_ref, node_ref, g_ref, w_ref, b_ref, out_ref):
    acc = jnp.dot(node_ref[...], w_ref[2 * D_EDGE:2 * D_EDGE + D_NODE, :],
                  preferred_element_type=jnp.float32)
    acc += jnp.dot(rec_ref[...], w_ref[:D_EDGE, :],
                   preferred_element_type=jnp.float32)
    acc += jnp.dot(sen_ref[...], w_ref[D_EDGE:2 * D_EDGE, :],
                   preferred_element_type=jnp.float32)
    acc += jnp.dot(g_ref[...], w_ref[2 * D_EDGE + D_NODE:, :],
                   preferred_element_type=jnp.float32)
    out_ref[...] = acc + b_ref[...]


def _matmul(rec, sen, node, g2, w, b2):
    d_in = 2 * D_EDGE + D_NODE + D_GLOBAL
    return pl.pallas_call(
        _mm_body,
        grid=(N_NODES // _BM,),
        in_specs=[
            pl.BlockSpec((_BM, D_EDGE), lambda i: (i, 0)),
            pl.BlockSpec((_BM, D_EDGE), lambda i: (i, 0)),
            pl.BlockSpec((_BM, D_NODE), lambda i: (i, 0)),
            pl.BlockSpec((1, D_GLOBAL), lambda i: (0, 0)),
            pl.BlockSpec((d_in, D_NODE), lambda i: (0, 0)),
            pl.BlockSpec((1, D_NODE), lambda i: (0, 0)),
        ],
        out_specs=pl.BlockSpec((_BM, D_NODE), lambda i: (i, 0)),
        out_shape=jax.ShapeDtypeStruct((N_NODES, D_NODE), jnp.float32),
    )(rec, sen, node, g2, w, b2)


def kernel(node_attributes, edge_attributes, global_attributes, edge_index, W, b):
    dst_r = edge_index[1].reshape(_NROWS, _IDXW)
    src_r = edge_index[0].reshape(_NROWS, _IDXW)
    rec, sen = _seg(edge_attributes, dst_r, src_r)
    return _matmul(rec, sen, node_attributes,
                   global_attributes.reshape(1, D_GLOBAL), W,
                   b.reshape(1, D_NODE))
